# baseline (speedup 1.0000x reference)
import functools

import jax
import jax.numpy as jnp
from jax import lax
from jax.experimental import pallas as pl
from jax.experimental.pallas import tpu as pltpu
from jax.experimental.pallas import tpu_sc as plsc

_NM = 4
_NB = 100000
_D = 64
_B = 16384
_NW = 32
_BPW = _B // _NW
_CH = 128
_NCH = _BPW // _CH

_mesh = plsc.VectorSubcoreMesh(core_axis_name="c", subcore_axis_name="s")


@functools.partial(
    pl.kernel,
    out_type=jax.ShapeDtypeStruct((_NM, _B, 2 * _D), jnp.float32),
    mesh=_mesh,
    compiler_params=pltpu.CompilerParams(use_tc_tiling_on_sc=False),
    scratch_types=[
        pltpu.VMEM((_NM, _NCH, _CH), jnp.int32),
        pltpu.VMEM((_CH, _D), jnp.float32),
        pltpu.VMEM((_CH, _D), jnp.float32),
        pltpu.VMEM((_CH, 2 * _D), jnp.float32),
        pltpu.SemaphoreType.DMA,
        pltpu.SemaphoreType.DMA,
    ],
)
def _sigmoid_boxes_sc(idx_hbm, w_hbm, W_hbm, out_hbm, idx_v, wrow, Wrow, zZ,
                      sem_w, sem_W):
    wid = lax.axis_index("s") * 2 + lax.axis_index("c")
    base = wid * _BPW
    pltpu.sync_copy(idx_hbm.at[wid], idx_v)

    for m in range(_NM):
        for j in range(_NCH):
            cw = pltpu.async_copy(w_hbm.at[idx_v.at[m, j]], wrow, sem_w)
            cW = pltpu.async_copy(W_hbm.at[idx_v.at[m, j]], Wrow, sem_W)
            cw.wait()
            cW.wait()

            def row_body(r, carry):
                for g in range(_D // 16):
                    sl = pl.ds(g * 16, 16)
                    x = wrow[r, sl]
                    y = Wrow[r, sl]
                    z = 1.0 / (1.0 + jnp.exp(-x))
                    s = 1.0 / (1.0 + jnp.exp(-y))
                    zZ[r, sl] = z
                    zZ[r, pl.ds(_D + g * 16, 16)] = z + s - s * z
                return carry

            lax.fori_loop(0, _CH, row_body, 0)
            pltpu.sync_copy(zZ, out_hbm.at[m, pl.ds(base + j * _CH, _CH)])


def kernel(box_indices, w, W):
    idx = box_indices.astype(jnp.int32).reshape(_NW, _NCH, _CH)
    offs = (jnp.arange(_NM, dtype=jnp.int32) * _NB).reshape(1, _NM, 1, 1)
    idx_all = idx[:, None, :, :] + offs
    w2 = w.reshape(_NM * _NB, _D)
    W2 = W.reshape(_NM * _NB, _D)
    out = _sigmoid_boxes_sc(idx_all, w2, W2)
    return out.reshape(_NM, _B, 2, _D)


# speedup vs baseline: 1.3317x; 1.3317x over previous
import functools

import jax
import jax.numpy as jnp
from jax import lax
from jax.experimental import pallas as pl
from jax.experimental.pallas import tpu as pltpu
from jax.experimental.pallas import tpu_sc as plsc

_NM = 4
_NB = 100000
_D = 64
_B = 16384
_NW = 32
_BPW = _B // _NW
_CH = 128
_NST = _NM * (_BPW // _CH)

_mesh = plsc.VectorSubcoreMesh(core_axis_name="c", subcore_axis_name="s")


@functools.partial(
    pl.kernel,
    out_type=jax.ShapeDtypeStruct((_NM, _B, 2 * _D), jnp.float32),
    mesh=_mesh,
    compiler_params=pltpu.CompilerParams(use_tc_tiling_on_sc=False),
    scratch_types=[
        pltpu.VMEM((_NST, _CH), jnp.int32),
        pltpu.VMEM((2, _CH, _D), jnp.float32),
        pltpu.VMEM((2, _CH, _D), jnp.float32),
        pltpu.VMEM((2, _CH, 2 * _D), jnp.float32),
        pltpu.SemaphoreType.DMA,
        pltpu.SemaphoreType.DMA,
        pltpu.SemaphoreType.DMA,
        pltpu.SemaphoreType.DMA,
    ],
)
def _sigmoid_boxes_sc(idx_hbm, w_hbm, W_hbm, out_hbm, idx_v, wrow, Wrow, zZ,
                      sem_g0, sem_g1, sem_o0, sem_o1):
    wid = lax.axis_index("s") * 2 + lax.axis_index("c")
    base = wid * _BPW
    pltpu.sync_copy(idx_hbm.at[wid], idx_v)

    sems_g = (sem_g0, sem_g1)
    sems_o = (sem_o0, sem_o1)
    gather_d = {}
    out_d = {}

    def start_gather(s):
        b = s % 2
        gather_d[s] = (
            pltpu.async_copy(w_hbm.at[idx_v.at[s]], wrow.at[b], sems_g[b]),
            pltpu.async_copy(W_hbm.at[idx_v.at[s]], Wrow.at[b], sems_g[b]),
        )

    start_gather(0)
    for s in range(_NST):
        b = s % 2
        if s + 1 < _NST:
            start_gather(s + 1)
        for c in gather_d.pop(s):
            c.wait()
        if s >= 2:
            out_d.pop(s - 2).wait()

        @plsc.parallel_loop(0, _CH, unroll=4)
        def _row(r):
            for g in range(_D // 16):
                sl = pl.ds(g * 16, 16)
                a = jnp.exp(-wrow[b, r, sl])
                e = jnp.exp(-Wrow[b, r, sl])
                ap = a + 1.0
                ep = e + 1.0
                rr = 1.0 / (ap * ep)
                zZ[b, r, sl] = ep * rr
                zZ[b, r, pl.ds(_D + g * 16, 16)] = (a + e + 1.0) * rr

        m, j = divmod(s, _NST // _NM)
        out_d[s] = pltpu.async_copy(
            zZ.at[b], out_hbm.at[m, pl.ds(base + j * _CH, _CH)], sems_o[b])

    out_d.pop(_NST - 2).wait()
    out_d.pop(_NST - 1).wait()


def kernel(box_indices, w, W):
    idx = box_indices.astype(jnp.int32).reshape(_NW, 1, _NM, _CH)
    offs = (jnp.arange(_NM, dtype=jnp.int32) * _NB).reshape(1, _NM, 1, 1)
    idx_all = (idx + offs).reshape(_NW, _NST, _CH)
    w2 = w.reshape(_NM * _NB, _D)
    W2 = W.reshape(_NM * _NB, _D)
    out = _sigmoid_boxes_sc(idx_all, w2, W2)
    return out.reshape(_NM, _B, 2, _D)
